# Initial kernel scaffold; baseline (speedup 1.0000x reference)
import jax
import jax.numpy as jnp
from jax.experimental import pallas as pl


def kernel(x, mask, w_s, w_f, mlp_w, mlp_b):
    raise NotImplementedError("write your pallas kernel here")


# speedup vs baseline: 1.0865x; 1.0865x over previous
import functools

import jax
import jax.numpy as jnp
from jax import lax
from jax.experimental import pallas as pl
from jax.experimental.pallas import tpu as pltpu
from jax.experimental.pallas import tpu_sc as plsc

B, N, D, K = 8, 2048, 256, 1024
RC = 256
NC, NS, L = 2, 16, 16
NW = NC * NS
TPB = NW // B
RPT = K // TPB
G = 16


def _score_body(mask_ref, x_ref, ws_ref, wf_ref, par_ref, out_ref):
    m = mask_ref[0]
    t1 = jnp.dot(m, ws_ref[:, :], preferred_element_type=jnp.float32)
    s1 = jnp.abs(t1.astype(jnp.bfloat16)).astype(jnp.float32)
    t2 = jnp.dot(x_ref[0], wf_ref[:, :], preferred_element_type=jnp.float32)
    s2 = jnp.abs(t2.astype(jnp.bfloat16)).astype(jnp.float32)
    a = par_ref[0, 0]
    bco = par_ref[0, 1]
    c = par_ref[0, 2]
    z = s1 * a + s2 * bco + c
    out_ref[0] = 1.0 / (1.0 + jnp.exp(-z))


def _stage_a(mask, x, w_s, w_f, params):
    nrc = N // RC
    return pl.pallas_call(
        _score_body,
        grid=(B, nrc),
        in_specs=[
            pl.BlockSpec((1, RC, N), lambda b, rc: (b, rc, 0)),
            pl.BlockSpec((1, RC, D), lambda b, rc: (b, rc, 0)),
            pl.BlockSpec((N, 1), lambda b, rc: (0, 0)),
            pl.BlockSpec((D, 1), lambda b, rc: (0, 0)),
            pl.BlockSpec((1, 3), lambda b, rc: (0, 0)),
        ],
        out_specs=pl.BlockSpec((1, RC, 1), lambda b, rc: (b * nrc + rc, 0, 0)),
        out_shape=jax.ShapeDtypeStruct((B * nrc, RC, 1), jnp.float32),
    )(mask, x, w_s, w_f, params)


def _cumsum2d(v):
    lane = lax.broadcasted_iota(jnp.int32, (16, 128), 1)
    for d in (1, 2, 4, 8, 16, 32, 64):
        v = v + jnp.where(lane >= d, pltpu.roll(v, d, 1), 0)
    rowt = v[:, 127:128]
    sub = lax.broadcasted_iota(jnp.int32, (16, 1), 0)
    rt = rowt
    for d in (1, 2, 4, 8):
        rt = rt + jnp.where(sub >= d, pltpu.roll(rt, d, 0), 0)
    return v + (rt - rowt)


def _select_body(s_ref, pos_ref):
    s = s_ref[0]
    si = lax.bitcast_convert_type(s, jnp.int32)

    def bs(_, lohi):
        lo, hi = lohi
        mid = (lo + hi) // 2
        cnt = jnp.sum((si >= mid).astype(jnp.int32))
        p = cnt >= K
        return jnp.where(p, mid, lo), jnp.where(p, hi, mid)

    lo, _ = lax.fori_loop(0, 30, bs, (jnp.int32(0), jnp.int32(1 << 30)))
    t = lo
    gt = si > t
    need = K - jnp.sum(gt.astype(jnp.int32))
    eq = si == t
    cum_eq = _cumsum2d(eq.astype(jnp.int32))
    sel = gt | (eq & (cum_eq <= need))
    cum_sel = _cumsum2d(sel.astype(jnp.int32))
    pos_ref[0] = jnp.where(sel, cum_sel - 1, -1)


def _stage_b(scores16):
    return pl.pallas_call(
        _select_body,
        grid=(B,),
        in_specs=[pl.BlockSpec((1, 16, 128), lambda b: (b, 0, 0))],
        out_specs=pl.BlockSpec((1, 16, 128), lambda b: (b, 0, 0)),
        out_shape=jax.ShapeDtypeStruct((B, 16, 128), jnp.int32),
    )(scores16)


def _pool_body(mask_hbm, x_hbm, sc_hbm, pos_hbm,
               mp_hbm, xp_hbm, oidx_hbm,
               pos_v, scall_v, idx_v, ssel_v, gidx_v,
               rows_v, out_v, xrow_v, xout_v, sem, semx):
    cid = lax.axis_index("c")
    sid = lax.axis_index("s")
    wid = sid * NC + cid
    b = wid // TPB
    row0 = (wid % TPB) * RPT

    pltpu.sync_copy(pos_hbm.at[pl.ds(b * N, N)], pos_v)
    pltpu.sync_copy(sc_hbm.at[pl.ds(b * N, N)], scall_v)

    lanes = lax.iota(jnp.int32, L)

    def comp(ch, _):
        off = pl.multiple_of(ch * L, L)
        pv = pos_v[pl.ds(off, L)]
        m = pv >= 0
        pvc = jnp.where(m, pv, 0)
        plsc.store_scatter(idx_v, [pvc], lanes + ch * L, mask=m)
        plsc.store_scatter(ssel_v, [pvc], scall_v[pl.ds(off, L)], mask=m)
        return 0

    lax.fori_loop(0, N // L, comp, 0)

    @pl.when(wid % TPB == 0)
    def _():
        pltpu.sync_copy(idx_v, oidx_hbm.at[pl.ds(b * K, K)])

    def gix(ch, _):
        src = pl.multiple_of(row0 + ch * L, L)
        dst = pl.multiple_of(ch * L, L)
        gidx_v[pl.ds(dst, L)] = idx_v[pl.ds(src, L)] + b * N
        return 0

    lax.fori_loop(0, RPT // L, gix, 0)

    def blk(i, _):
        base = pl.multiple_of(i * G, G)
        g = b * K + row0 + base
        pltpu.async_copy(mask_hbm.at[gidx_v.at[pl.ds(base, G)]], rows_v, sem).wait()
        pltpu.async_copy(x_hbm.at[gidx_v.at[pl.ds(base, G)]], xrow_v, semx).wait()

        def ccol(ci, _):
            off = pl.multiple_of(ci * L, L)
            colv = idx_v[pl.ds(off, L)]
            for r in range(G):
                rv = jnp.full((L,), r, jnp.int32)
                out_v[r, pl.ds(off, L)] = plsc.load_gather(rows_v, [rv, colv])
            return 0

        lax.fori_loop(0, K // L, ccol, 0)

        sch = ssel_v[pl.ds(pl.multiple_of(row0 + base, L), L)]
        for r in range(G):
            sval = jnp.sum(jnp.where(lanes == r, sch, 0.0))
            for dc in range(D // L):
                xout_v[r, pl.ds(dc * L, L)] = xrow_v[r, pl.ds(dc * L, L)] * sval

        pltpu.sync_copy(out_v, mp_hbm.at[pl.ds(g, G)])
        pltpu.sync_copy(xout_v, xp_hbm.at[pl.ds(g, G)])
        return 0

    lax.fori_loop(0, RPT // G, blk, 0)


def _stage_c(mask_flat, x_flat, scores_flat, pos_flat):
    mesh = plsc.VectorSubcoreMesh(core_axis_name="c", subcore_axis_name="s")
    return pl.kernel(
        _pool_body,
        out_type=[
            jax.ShapeDtypeStruct((B * K, K), jnp.float32),
            jax.ShapeDtypeStruct((B * K, D), jnp.float32),
            jax.ShapeDtypeStruct((B * K,), jnp.int32),
        ],
        mesh=mesh,
        compiler_params=pltpu.CompilerParams(needs_layout_passes=False),
        scratch_types=[
            pltpu.VMEM((N,), jnp.int32),
            pltpu.VMEM((N,), jnp.float32),
            pltpu.VMEM((K,), jnp.int32),
            pltpu.VMEM((K,), jnp.float32),
            pltpu.VMEM((RPT,), jnp.int32),
            pltpu.VMEM((G, N), jnp.float32),
            pltpu.VMEM((G, K), jnp.float32),
            pltpu.VMEM((G, D), jnp.float32),
            pltpu.VMEM((G, D), jnp.float32),
            pltpu.SemaphoreType.DMA,
            pltpu.SemaphoreType.DMA,
        ],
    )(mask_flat, x_flat, scores_flat, pos_flat)


def kernel(x, mask, w_s, w_f, mlp_w, mlp_b):
    wmlp = lax.reduce_precision(mlp_w.reshape(-1), exponent_bits=8,
                                mantissa_bits=7)
    params = jnp.concatenate([wmlp, mlp_b.reshape(-1)]).reshape(1, 3)
    scores3d = _stage_a(mask, x, w_s, w_f, params)
    scores = scores3d.reshape(B, N)
    pos = _stage_b(scores.reshape(B, 16, 128))
    mpool, xpool, idx = _stage_c(
        mask.reshape(B * N, N),
        x.reshape(B * N, D),
        scores.reshape(B * N),
        pos.reshape(B * N),
    )
    return (xpool.reshape(B, K, D), mpool.reshape(B, K, K), idx.reshape(B, K))


# speedup vs baseline: 1.4453x; 1.3302x over previous
import functools

import jax
import jax.numpy as jnp
from jax import lax
from jax.experimental import pallas as pl
from jax.experimental.pallas import tpu as pltpu
from jax.experimental.pallas import tpu_sc as plsc

B, N, D, K = 8, 2048, 256, 1024
RC = 512
NC, NS, L = 2, 16, 16
NW = NC * NS
TPB = NW // B
RPT = K // TPB
G = 16


def _score_body(mask_ref, x_ref, ws_ref, wf_ref, par_ref, out_ref):
    m = mask_ref[0]
    t1 = jnp.dot(m, ws_ref[:, :], preferred_element_type=jnp.float32)
    s1 = jnp.abs(t1.astype(jnp.bfloat16)).astype(jnp.float32)
    t2 = jnp.dot(x_ref[0], wf_ref[:, :], preferred_element_type=jnp.float32)
    s2 = jnp.abs(t2.astype(jnp.bfloat16)).astype(jnp.float32)
    a = par_ref[0, 0]
    bco = par_ref[0, 1]
    c = par_ref[0, 2]
    z = s1 * a + s2 * bco + c
    out_ref[0] = 1.0 / (1.0 + jnp.exp(-z))


def _stage_a(mask, x, w_s, w_f, params):
    nrc = N // RC
    return pl.pallas_call(
        _score_body,
        grid=(B, nrc),
        in_specs=[
            pl.BlockSpec((1, RC, N), lambda b, rc: (b, rc, 0)),
            pl.BlockSpec((1, RC, D), lambda b, rc: (b, rc, 0)),
            pl.BlockSpec((N, 1), lambda b, rc: (0, 0)),
            pl.BlockSpec((D, 1), lambda b, rc: (0, 0)),
            pl.BlockSpec((1, 3), lambda b, rc: (0, 0)),
        ],
        out_specs=pl.BlockSpec((1, RC, 1), lambda b, rc: (b * nrc + rc, 0, 0)),
        out_shape=jax.ShapeDtypeStruct((B * nrc, RC, 1), jnp.float32),
    )(mask, x, w_s, w_f, params)


def _cumsum2d(v):
    lane = lax.broadcasted_iota(jnp.int32, (16, 128), 1)
    for d in (1, 2, 4, 8, 16, 32, 64):
        v = v + jnp.where(lane >= d, pltpu.roll(v, d, 1), 0)
    rowt = v[:, 127:128]
    sub = lax.broadcasted_iota(jnp.int32, (16, 1), 0)
    rt = rowt
    for d in (1, 2, 4, 8):
        rt = rt + jnp.where(sub >= d, pltpu.roll(rt, d, 0), 0)
    return v + (rt - rowt)


def _select_body(s_ref, pos_ref):
    s = s_ref[0]
    si = lax.bitcast_convert_type(s, jnp.int32)

    def bs(_, lohi):
        lo, hi = lohi
        mid = (lo + hi) // 2
        cnt = jnp.sum((si >= mid).astype(jnp.int32))
        p = cnt >= K
        return jnp.where(p, mid, lo), jnp.where(p, hi, mid)

    lo, _ = lax.fori_loop(0, 30, bs, (jnp.int32(0), jnp.int32(1 << 30)))
    t = lo
    gt = si > t
    need = K - jnp.sum(gt.astype(jnp.int32))
    eq = si == t
    cum_eq = _cumsum2d(eq.astype(jnp.int32))
    sel = gt | (eq & (cum_eq <= need))
    cum_sel = _cumsum2d(sel.astype(jnp.int32))
    pos_ref[0] = jnp.where(sel, cum_sel - 1, -1)


def _stage_b(scores16):
    return pl.pallas_call(
        _select_body,
        grid=(B,),
        in_specs=[pl.BlockSpec((1, 16, 128), lambda b: (b, 0, 0))],
        out_specs=pl.BlockSpec((1, 16, 128), lambda b: (b, 0, 0)),
        out_shape=jax.ShapeDtypeStruct((B, 16, 128), jnp.int32),
    )(scores16)


def _pool_body(mask_hbm, x_hbm, sc_hbm, pos_hbm,
               mp_hbm, xp_hbm, oidx_hbm,
               pos_v, scall_v, idx_v, ssel_v, gidx_v,
               rows_v, out_v, xrow_v, xout_v,
               sga0, sga1, sgx0, sgx1, semw):
    sems = (sga0, sga1)
    semx = (sgx0, sgx1)
    cid = lax.axis_index("c")
    sid = lax.axis_index("s")
    wid = sid * NC + cid
    b = wid // TPB
    row0 = (wid % TPB) * RPT

    pltpu.sync_copy(pos_hbm.at[pl.ds(b * N, N)], pos_v)
    pltpu.sync_copy(sc_hbm.at[pl.ds(b * N, N)], scall_v)

    lanes = lax.iota(jnp.int32, L)

    def comp(ch, _):
        off = pl.multiple_of(ch * L, L)
        pv = pos_v[pl.ds(off, L)]
        m = pv >= 0
        pvc = jnp.where(m, pv, 0)
        plsc.store_scatter(idx_v, [pvc], lanes + ch * L, mask=m)
        plsc.store_scatter(ssel_v, [pvc], scall_v[pl.ds(off, L)], mask=m)
        return 0

    lax.fori_loop(0, N // L, comp, 0)

    @pl.when(wid % TPB == 0)
    def _():
        pltpu.sync_copy(idx_v, oidx_hbm.at[pl.ds(b * K, K)])

    def gix(ch, _):
        src = pl.multiple_of(row0 + ch * L, L)
        dst = pl.multiple_of(ch * L, L)
        gidx_v[pl.ds(dst, L)] = idx_v[pl.ds(src, L)] + b * N
        return 0

    lax.fori_loop(0, RPT // L, gix, 0)

    NB = RPT // G

    def start_gather(i, p):
        base = pl.multiple_of(i * G, G)
        pltpu.async_copy(mask_hbm.at[gidx_v.at[pl.ds(base, G)]],
                         rows_v.at[p], sems[p])
        pltpu.async_copy(x_hbm.at[gidx_v.at[pl.ds(base, G)]],
                         xrow_v.at[p], semx[p])

    start_gather(0, 0)

    def process(i, p):
        base = pl.multiple_of(i * G, G)
        g = b * K + row0 + base

        @pl.when(i + 1 < NB)
        def _():
            start_gather(i + 1, 1 - p)

        @pl.when(i >= 2)
        def _():
            gp = g - 2 * G
            pltpu.make_async_copy(out_v.at[p], mp_hbm.at[pl.ds(gp, G)], semw).wait()
            pltpu.make_async_copy(xout_v.at[p], xp_hbm.at[pl.ds(gp, G)], semw).wait()

        pltpu.make_async_copy(mask_hbm.at[gidx_v.at[pl.ds(base, G)]],
                              rows_v.at[p], sems[p]).wait()

        def ccol(ci, _):
            off = pl.multiple_of(ci * L, L)
            colv = idx_v[pl.ds(off, L)]
            pv = jnp.full((L,), p, jnp.int32)
            for r in range(G):
                rv = jnp.full((L,), r, jnp.int32)
                out_v[p, r, pl.ds(off, L)] = plsc.load_gather(rows_v, [pv, rv, colv])
            return 0

        lax.fori_loop(0, K // L, ccol, 0)

        pltpu.make_async_copy(x_hbm.at[gidx_v.at[pl.ds(base, G)]],
                              xrow_v.at[p], semx[p]).wait()
        sch = ssel_v[pl.ds(pl.multiple_of(row0 + base, L), L)]
        for r in range(G):
            sval = jnp.sum(jnp.where(lanes == r, sch, 0.0))
            for dc in range(D // L):
                xout_v[p, r, pl.ds(dc * L, L)] = xrow_v[p, r, pl.ds(dc * L, L)] * sval

        pltpu.async_copy(out_v.at[p], mp_hbm.at[pl.ds(g, G)], semw)
        pltpu.async_copy(xout_v.at[p], xp_hbm.at[pl.ds(g, G)], semw)

    def superblk(j, _):
        i = j * 2
        process(i, 0)
        process(i + 1, 1)
        return 0

    lax.fori_loop(0, NB // 2, superblk, 0)

    for i in (NB - 2, NB - 1):
        p = i % 2
        g = b * K + row0 + i * G
        pltpu.make_async_copy(out_v.at[p], mp_hbm.at[pl.ds(g, G)], semw).wait()
        pltpu.make_async_copy(xout_v.at[p], xp_hbm.at[pl.ds(g, G)], semw).wait()


def _stage_c(mask_flat, x_flat, scores_flat, pos_flat):
    mesh = plsc.VectorSubcoreMesh(core_axis_name="c", subcore_axis_name="s")
    return pl.kernel(
        _pool_body,
        out_type=[
            jax.ShapeDtypeStruct((B * K, K), jnp.float32),
            jax.ShapeDtypeStruct((B * K, D), jnp.float32),
            jax.ShapeDtypeStruct((B * K,), jnp.int32),
        ],
        mesh=mesh,
        compiler_params=pltpu.CompilerParams(needs_layout_passes=False),
        scratch_types=[
            pltpu.VMEM((N,), jnp.int32),
            pltpu.VMEM((N,), jnp.float32),
            pltpu.VMEM((K,), jnp.int32),
            pltpu.VMEM((K,), jnp.float32),
            pltpu.VMEM((RPT,), jnp.int32),
            pltpu.VMEM((2, G, N), jnp.float32),
            pltpu.VMEM((2, G, K), jnp.float32),
            pltpu.VMEM((2, G, D), jnp.float32),
            pltpu.VMEM((2, G, D), jnp.float32),
            pltpu.SemaphoreType.DMA,
            pltpu.SemaphoreType.DMA,
            pltpu.SemaphoreType.DMA,
            pltpu.SemaphoreType.DMA,
            pltpu.SemaphoreType.DMA,
        ],
    )(mask_flat, x_flat, scores_flat, pos_flat)


def kernel(x, mask, w_s, w_f, mlp_w, mlp_b):
    wmlp = lax.reduce_precision(mlp_w.reshape(-1), exponent_bits=8,
                                mantissa_bits=7)
    params = jnp.concatenate([wmlp, mlp_b.reshape(-1)]).reshape(1, 3)
    scores3d = _stage_a(mask, x, w_s, w_f, params)
    scores = scores3d.reshape(B, N)
    pos = _stage_b(scores.reshape(B, 16, 128))
    mpool, xpool, idx = _stage_c(
        mask.reshape(B * N, N),
        x.reshape(B * N, D),
        scores.reshape(B * N),
        pos.reshape(B * N),
    )
    return (xpool.reshape(B, K, D), mpool.reshape(B, K, K), idx.reshape(B, K))


# speedup vs baseline: 1.4514x; 1.0042x over previous
import jax
import jax.numpy as jnp
from jax import lax
from jax.experimental import pallas as pl
from jax.experimental.pallas import tpu as pltpu
from jax.experimental.pallas import tpu_sc as plsc

B, N, D, K = 8, 2048, 256, 1024
RC = 512
NC, NS, L = 2, 16, 16
NW = NC * NS
TPB = NW // B
RPT = K // TPB
G = 16


def _score_body(mask_ref, x_ref, ws_ref, wf_ref, par_ref, out_ref):
    m = mask_ref[0]
    t1 = jnp.dot(m, ws_ref[:, :], preferred_element_type=jnp.float32)
    s1 = jnp.abs(t1.astype(jnp.bfloat16)).astype(jnp.float32)
    t2 = jnp.dot(x_ref[0], wf_ref[:, :], preferred_element_type=jnp.float32)
    s2 = jnp.abs(t2.astype(jnp.bfloat16)).astype(jnp.float32)
    a = par_ref[0, 0]
    bco = par_ref[0, 1]
    c = par_ref[0, 2]
    z = s1 * a + s2 * bco + c
    out_ref[0] = 1.0 / (1.0 + jnp.exp(-z))


def _stage_a(mask, x, w_s, w_f, params):
    nrc = N // RC
    return pl.pallas_call(
        _score_body,
        grid=(B, nrc),
        in_specs=[
            pl.BlockSpec((1, RC, N), lambda b, rc: (b, rc, 0)),
            pl.BlockSpec((1, RC, D), lambda b, rc: (b, rc, 0)),
            pl.BlockSpec((N, 1), lambda b, rc: (0, 0)),
            pl.BlockSpec((D, 1), lambda b, rc: (0, 0)),
            pl.BlockSpec((1, 3), lambda b, rc: (0, 0)),
        ],
        out_specs=pl.BlockSpec((1, RC, 1), lambda b, rc: (b * nrc + rc, 0, 0)),
        out_shape=jax.ShapeDtypeStruct((B * nrc, RC, 1), jnp.float32),
    )(mask, x, w_s, w_f, params)


def _cumsum2d(v):
    lane = lax.broadcasted_iota(jnp.int32, (16, 128), 1)
    for d in (1, 2, 4, 8, 16, 32, 64):
        v = v + jnp.where(lane >= d, pltpu.roll(v, d, 1), 0)
    rowt = v[:, 127:128]
    sub = lax.broadcasted_iota(jnp.int32, (16, 1), 0)
    rt = rowt
    for d in (1, 2, 4, 8):
        rt = rt + jnp.where(sub >= d, pltpu.roll(rt, d, 0), 0)
    return v + (rt - rowt)


def _select_body(s_ref, pos_ref):
    s = s_ref[0]
    si = lax.bitcast_convert_type(s, jnp.int32)

    def bs(_, lohi):
        lo, hi = lohi
        mid = (lo + hi) // 2
        cnt = jnp.sum((si >= mid).astype(jnp.int32))
        p = cnt >= K
        return jnp.where(p, mid, lo), jnp.where(p, hi, mid)

    lo, _ = lax.fori_loop(0, 30, bs, (jnp.int32(0), jnp.int32(1 << 30)))
    t = lo
    gt = si > t
    need = K - jnp.sum(gt.astype(jnp.int32))
    eq = si == t
    cum_eq = _cumsum2d(eq.astype(jnp.int32))
    sel = gt | (eq & (cum_eq <= need))
    cum_sel = _cumsum2d(sel.astype(jnp.int32))
    pos_ref[0] = jnp.where(sel, cum_sel - 1, -1)


def _stage_b(scores16):
    return pl.pallas_call(
        _select_body,
        grid=(B,),
        in_specs=[pl.BlockSpec((1, 16, 128), lambda b: (b, 0, 0))],
        out_specs=pl.BlockSpec((1, 16, 128), lambda b: (b, 0, 0)),
        out_shape=jax.ShapeDtypeStruct((B, 16, 128), jnp.int32),
    )(scores16)


def _pool_body(mask_hbm, x_hbm, sc_hbm, pos_hbm,
               mp_hbm, xp_hbm, oidx_hbm,
               pos_v, scall_v, idx_v, ssel_v, gidx_v,
               rows_v, out_v, xrow_v, xout_v,
               sga0, sga1, sgx0, sgx1, semw):
    sems = (sga0, sga1)
    semx = (sgx0, sgx1)
    cid = lax.axis_index("c")
    sid = lax.axis_index("s")
    wid = sid * NC + cid
    b = wid // TPB
    row0 = (wid % TPB) * RPT

    pltpu.sync_copy(pos_hbm.at[pl.ds(b * N, N)], pos_v)
    pltpu.sync_copy(sc_hbm.at[pl.ds(b * N, N)], scall_v)

    lanes = lax.iota(jnp.int32, L)

    def comp(ch, _):
        off = pl.multiple_of(ch * L, L)
        pv = pos_v[pl.ds(off, L)]
        m = pv >= 0
        pvc = jnp.where(m, pv, 0)
        plsc.store_scatter(idx_v, [pvc], lanes + ch * L, mask=m)
        plsc.store_scatter(ssel_v, [pvc], scall_v[pl.ds(off, L)], mask=m)
        return 0

    lax.fori_loop(0, N // L, comp, 0)

    @pl.when(wid % TPB == 0)
    def _():
        pltpu.sync_copy(idx_v, oidx_hbm.at[pl.ds(b * K, K)])

    def gix(ch, _):
        src = pl.multiple_of(row0 + ch * L, L)
        dst = pl.multiple_of(ch * L, L)
        gidx_v[pl.ds(dst, L)] = idx_v[pl.ds(src, L)] + b * N
        return 0

    lax.fori_loop(0, RPT // L, gix, 0)

    NB = RPT // G

    def start_gather(i, p):
        base = pl.multiple_of(i * G, G)
        pltpu.async_copy(mask_hbm.at[gidx_v.at[pl.ds(base, G)]],
                         rows_v.at[p], sems[p])
        pltpu.async_copy(x_hbm.at[gidx_v.at[pl.ds(base, G)]],
                         xrow_v.at[p], semx[p])

    start_gather(0, 0)

    def process(i, p):
        base = pl.multiple_of(i * G, G)
        g = b * K + row0 + base

        @pl.when(i + 1 < NB)
        def _():
            start_gather(i + 1, 1 - p)

        @pl.when(i >= 2)
        def _():
            gp = g - 2 * G
            pltpu.make_async_copy(out_v.at[p], mp_hbm.at[pl.ds(gp, G)], semw).wait()
            pltpu.make_async_copy(xout_v.at[p], xp_hbm.at[pl.ds(gp, G)], semw).wait()

        pltpu.make_async_copy(mask_hbm.at[gidx_v.at[pl.ds(base, G)]],
                              rows_v.at[p], sems[p]).wait()

        def ccol(ci, _):
            off = pl.multiple_of(ci * L, L)
            colv = idx_v[pl.ds(off, L)]
            pv = jnp.full((L,), p, jnp.int32)
            for r in range(G):
                rv = jnp.full((L,), r, jnp.int32)
                out_v[p, r, pl.ds(off, L)] = plsc.load_gather(rows_v, [pv, rv, colv])
            return 0

        lax.fori_loop(0, K // L, ccol, 0)

        pltpu.make_async_copy(x_hbm.at[gidx_v.at[pl.ds(base, G)]],
                              xrow_v.at[p], semx[p]).wait()
        sch = ssel_v[pl.ds(pl.multiple_of(row0 + base, L), L)]
        for r in range(G):
            sval = jnp.sum(jnp.where(lanes == r, sch, 0.0))
            for dc in range(D // L):
                xout_v[p, r, pl.ds(dc * L, L)] = xrow_v[p, r, pl.ds(dc * L, L)] * sval

        pltpu.async_copy(out_v.at[p], mp_hbm.at[pl.ds(g, G)], semw)
        pltpu.async_copy(xout_v.at[p], xp_hbm.at[pl.ds(g, G)], semw)

    def superblk(j, _):
        i = j * 2
        process(i, 0)
        process(i + 1, 1)
        return 0

    lax.fori_loop(0, NB // 2, superblk, 0)

    for i in (NB - 2, NB - 1):
        p = i % 2
        g = b * K + row0 + i * G
        pltpu.make_async_copy(out_v.at[p], mp_hbm.at[pl.ds(g, G)], semw).wait()
        pltpu.make_async_copy(xout_v.at[p], xp_hbm.at[pl.ds(g, G)], semw).wait()


def _stage_c(mask_flat, x_flat, scores_flat, pos_flat):
    mesh = plsc.VectorSubcoreMesh(core_axis_name="c", subcore_axis_name="s")
    return pl.kernel(
        _pool_body,
        out_type=[
            jax.ShapeDtypeStruct((B * K, K), jnp.float32),
            jax.ShapeDtypeStruct((B * K, D), jnp.float32),
            jax.ShapeDtypeStruct((B * K,), jnp.int32),
        ],
        mesh=mesh,
        compiler_params=pltpu.CompilerParams(needs_layout_passes=False),
        scratch_types=[
            pltpu.VMEM((N,), jnp.int32),
            pltpu.VMEM((N,), jnp.float32),
            pltpu.VMEM((K,), jnp.int32),
            pltpu.VMEM((K,), jnp.float32),
            pltpu.VMEM((RPT,), jnp.int32),
            pltpu.VMEM((2, G, N), jnp.float32),
            pltpu.VMEM((2, G, K), jnp.float32),
            pltpu.VMEM((2, G, D), jnp.float32),
            pltpu.VMEM((2, G, D), jnp.float32),
            pltpu.SemaphoreType.DMA,
            pltpu.SemaphoreType.DMA,
            pltpu.SemaphoreType.DMA,
            pltpu.SemaphoreType.DMA,
            pltpu.SemaphoreType.DMA,
        ],
    )(mask_flat, x_flat, scores_flat, pos_flat)


def kernel(x, mask, w_s, w_f, mlp_w, mlp_b):
    wmlp = lax.reduce_precision(mlp_w.reshape(-1), exponent_bits=8,
                                mantissa_bits=7)
    params = jnp.concatenate([wmlp, mlp_b.reshape(-1)]).reshape(1, 3)
    scores3d = _stage_a(mask, x, w_s, w_f, params)
    scores = scores3d.reshape(B, N)
    pos = _stage_b(scores.reshape(B, 16, 128))
    mpool, xpool, idx = _stage_c(
        mask.reshape(B * N, N),
        x.reshape(B * N, D),
        scores.reshape(B * N),
        pos.reshape(B * N),
    )
    return (xpool.reshape(B, K, D), mpool.reshape(B, K, K), idx.reshape(B, K))
